# baseline (speedup 1.0000x reference)
import functools

import jax
import jax.numpy as jnp
from jax import lax
from jax.experimental import pallas as pl
from jax.experimental.pallas import tpu as pltpu
from jax.experimental.pallas import tpu_sc as plsc

D = 1024
B = 16
S = 4096

_NC = 2
_NS = 16
_W = _NC * _NS
_RPW = S // _W
_ZR = 64


def _sc_body(data_hbm, len_hbm, off_hbm, out_hbm, len_v, off_v, zeros_v, sem):
    cid = lax.axis_index("c")
    sid = lax.axis_index("s")
    wid = sid * _NC + cid

    pltpu.sync_copy(len_hbm, len_v)
    pltpu.sync_copy(off_hbm, off_v)

    def _zchunk(j, _):
        zeros_v[pl.ds(j * 16, 16)] = jnp.zeros((16,), jnp.float32)
        return 0
    lax.fori_loop(0, (_ZR * D) // 16, _zchunk, 0)

    start = wid * _RPW

    def _batch(b, _):
        len_b = len_v[pl.ds(b * 16, 16)][0]
        off_b = off_v[pl.ds(b * 16, 16)][0]
        valid = jnp.clip(len_b - start, 0, _RPW)
        dbase = b * S + start
        sbase = off_b + start

        @pl.when(valid == _RPW)
        def _():
            pltpu.async_copy(
                data_hbm.at[pl.ds(sbase * D, _RPW * D)],
                out_hbm.at[pl.ds(dbase * D, _RPW * D)],
                sem,
            ).wait()

        @pl.when(valid < _RPW)
        def _():
            cur = jnp.int32(0)
            rem = valid
            for blk in (64, 32, 16, 8, 4, 2, 1):
                take = rem >= blk

                @pl.when(take)
                def _(cur=cur, blk=blk):
                    pltpu.async_copy(
                        data_hbm.at[pl.ds((sbase + cur) * D, blk * D)],
                        out_hbm.at[pl.ds((dbase + cur) * D, blk * D)],
                        sem,
                    ).wait()

                step = jnp.where(take, blk, 0)
                cur = cur + step
                rem = rem - step

            zcur = valid
            zrem = _RPW - valid
            for blk in (64, 64, 32, 16, 8, 4, 2, 1):
                ztake = zrem >= blk

                @pl.when(ztake)
                def _(zcur=zcur, blk=blk):
                    pltpu.async_copy(
                        zeros_v.at[pl.ds(0, blk * D)],
                        out_hbm.at[pl.ds((dbase + zcur) * D, blk * D)],
                        sem,
                    ).wait()

                zstep = jnp.where(ztake, blk, 0)
                zcur = zcur + zstep
                zrem = zrem - zstep

        return 0

    lax.fori_loop(0, B, _batch, 0)


def _mask_body(len_ref, out_ref):
    out_ref[...] = (
        lax.broadcasted_iota(jnp.int32, (B, S), 1) < len_ref[...]
    )


def kernel(data, lengths):
    mesh = plsc.VectorSubcoreMesh(core_axis_name="c", subcore_axis_name="s")
    scatter = pl.kernel(
        _sc_body,
        mesh=mesh,
        out_type=jax.ShapeDtypeStruct((B * S * D,), jnp.float32),
        scratch_types=[
            pltpu.VMEM((B * 16,), jnp.int32),
            pltpu.VMEM((B * 16,), jnp.int32),
            pltpu.VMEM((_ZR * D,), jnp.float32),
            pltpu.SemaphoreType.DMA,
        ],
    )
    offsets = (jnp.cumsum(lengths) - lengths).astype(jnp.int32)
    len_rep = jnp.repeat(lengths, 16)
    off_rep = jnp.repeat(offsets, 16)
    padded_flat = scatter(data.reshape(-1), len_rep, off_rep)

    mask = pl.pallas_call(
        _mask_body,
        out_shape=jax.ShapeDtypeStruct((B, S), jnp.bool_),
    )(lengths.reshape(B, 1))

    return padded_flat.reshape(B, S, D), mask


# speedup vs baseline: 28.1646x; 28.1646x over previous
import jax
import jax.numpy as jnp
from jax import lax
from jax.experimental import pallas as pl
from jax.experimental.pallas import tpu as pltpu
from jax.experimental.pallas import tpu_sc as plsc

D = 1024
B = 16
S = 4096

_NC = 2
_NS = 16
_W = _NC * _NS
_RPW = S // _W
_UR = 32
_UPB = _RPW // _UR
_ZR = 16
_NBANK = 3
_NUNIT = B * _UPB


def _sc_body(data_hbm, len_hbm, out_hbm,
             len_v16, len_v, off_v, s0, s1, s2, x0, x1, x2, zeros_v,
             i0, i1, i2, o0, o1, o2):
    slots = (s0, s1, s2)
    idxs = (x0, x1, x2)
    sin = (i0, i1, i2)
    sout = (o0, o1, o2)
    cid = lax.axis_index("c")
    sid = lax.axis_index("s")
    wid = sid * _NC + cid
    lane = lax.iota(jnp.int32, 16)

    pltpu.sync_copy(len_hbm, len_v16)
    lv = len_v16[...]
    x = lv
    for k in (1, 2, 4, 8):
        sh = x.at[jnp.clip(lane - k, 0, 15)].get(
            mode="promise_in_bounds")
        x = x + jnp.where(lane >= k, sh, 0)
    off = x - lv
    for b in range(B):
        bidx = jnp.full((16,), b, jnp.int32)
        len_v[pl.ds(b * 16, 16)] = lv.at[bidx].get(
            mode="promise_in_bounds")
        off_v[pl.ds(b * 16, 16)] = off.at[bidx].get(
            mode="promise_in_bounds")

    def unit_info(u):
        b = u // _UPB
        kk = u % _UPB
        len_b = len_v[pl.ds(b * 16, 16)][0]
        off_b = off_v[pl.ds(b * 16, 16)][0]
        pos = wid * _UR + kk * (_UR * _W)
        valid = jnp.clip(len_b - pos, 0, _UR)
        src = off_b + pos
        dst = b * S + pos
        return valid, src, dst

    def issue_in(u, sl):
        valid, src, dst = unit_info(u)

        @pl.when(valid > 0)
        def _():
            for c in range(_UR // 16):
                p = c * 16 + lane
                idxs[sl][pl.ds(c * 16, 16)] = jnp.where(
                    p < valid, src + p, 0)
            pltpu.async_copy(
                data_hbm.at[idxs[sl]], slots[sl], sin[sl])

    def consume(u, sl):
        valid, src, dst = unit_info(u)
        zero = valid == 0
        part = jnp.logical_and(valid > 0, valid < _UR)

        @pl.when(valid > 0)
        def _():
            pltpu.make_async_copy(
                data_hbm.at[idxs[sl]], slots[sl], sin[sl]).wait()

        @pl.when(part)
        def _():
            def _zt(r, _):
                def _zc(c, _):
                    slots[sl][r, pl.ds(c * 16, 16)] = jnp.zeros(
                        (16,), jnp.float32)
                    return 0
                return lax.fori_loop(0, D // 16, _zc, 0)
            lax.fori_loop(valid, _UR, _zt, 0)

        @pl.when(zero)
        def _():
            for h in range(_UR // _ZR):
                pltpu.async_copy(
                    zeros_v,
                    out_hbm.at[pl.ds(pl.multiple_of(dst + h * _ZR, 8),
                                     _ZR), :],
                    sout[sl])

        @pl.when(jnp.logical_not(zero))
        def _():
            pltpu.async_copy(
                slots[sl],
                out_hbm.at[pl.ds(pl.multiple_of(dst, 8), _UR), :], sout[sl])

    def wait_out(u, sl):
        valid, src, dst = unit_info(u)

        @pl.when(valid == 0)
        def _():
            for h in range(_UR // _ZR):
                pltpu.make_async_copy(
                    zeros_v, out_hbm.at[pl.ds(0, _ZR), :], sout[sl]).wait()

        @pl.when(valid > 0)
        def _():
            pltpu.make_async_copy(
                slots[sl], out_hbm.at[pl.ds(0, _UR), :], sout[sl]).wait()

    issue_in(0, 0)
    issue_in(1, 1)

    def _z(r, _):
        def _zc(c, _):
            zeros_v[r, pl.ds(c * 16, 16)] = jnp.zeros((16,), jnp.float32)
            return 0
        return lax.fori_loop(0, D // 16, _zc, 0)
    lax.fori_loop(0, _ZR, _z, 0)


    def _triple(q, _):
        for p in range(_NBANK):
            u = _NBANK * q + p
            consume(u, p)
            rb = (p + _NBANK - 1) % _NBANK

            @pl.when(u >= 1)
            def _(u=u, rb=rb):
                wait_out(u - 1, rb)

            @pl.when(u + 2 <= _NUNIT - 1)
            def _(u=u, rb=rb):
                issue_in(u + 2, rb)
        return 0

    lax.fori_loop(0, (_NUNIT - 1) // _NBANK, _triple, 0)

    consume(_NUNIT - 1, 0)
    wait_out(_NUNIT - 2, 2)
    wait_out(_NUNIT - 1, 0)


def _mask_body(len_ref, out_ref):
    out_ref[...] = (
        lax.broadcasted_iota(jnp.int32, (B, S), 1) < len_ref[...]
    )


def kernel(data, lengths):
    mesh = plsc.VectorSubcoreMesh(core_axis_name="c", subcore_axis_name="s")
    scatter = pl.kernel(
        _sc_body,
        mesh=mesh,
        out_type=jax.ShapeDtypeStruct((B * S, D), jnp.float32),
        scratch_types=[
            pltpu.VMEM((16,), jnp.int32),
            pltpu.VMEM((B * 16,), jnp.int32),
            pltpu.VMEM((B * 16,), jnp.int32),
            pltpu.VMEM((_UR, D), jnp.float32),
            pltpu.VMEM((_UR, D), jnp.float32),
            pltpu.VMEM((_UR, D), jnp.float32),
            pltpu.VMEM((_UR,), jnp.int32),
            pltpu.VMEM((_UR,), jnp.int32),
            pltpu.VMEM((_UR,), jnp.int32),
            pltpu.VMEM((_ZR, D), jnp.float32),
        ] + [pltpu.SemaphoreType.DMA] * 6,
    )
    padded_flat = scatter(data, lengths)

    mask = pl.pallas_call(
        _mask_body,
        out_shape=jax.ShapeDtypeStruct((B, S), jnp.bool_),
    )(lengths.reshape(B, 1))

    return padded_flat.reshape(B, S, D), mask
